# baseline (speedup 1.0000x reference)
import functools

import jax
import jax.numpy as jnp
from jax.experimental import pallas as pl
from jax.experimental.pallas import tpu as pltpu


def _router_body(chan_emb_ref, wr1_ref, br1_ref, wr2_ref, br2_ref, w1f_ref,
                 gates_ref, w1sum_ref, *, R, E):
    hidden = jnp.maximum(
        jax.lax.dot_general(
            chan_emb_ref[...], wr1_ref[...],
            (((1,), (0,)), ((), ())), preferred_element_type=jnp.float32,
        ) + br1_ref[...],
        0.0,
    )
    logits = jax.lax.dot_general(
        hidden, wr2_ref[...],
        (((1,), (0,)), ((), ())), preferred_element_type=jnp.float32,
    ) + br2_ref[...]
    m = jnp.max(logits, axis=-1, keepdims=True)
    ex = jnp.exp(logits - m)
    gates = ex / jnp.sum(ex, axis=-1, keepdims=True)
    gates_ref[...] = jnp.concatenate(
        [jnp.broadcast_to(gates[:, e:e + 1], gates.shape[:1] + (R,))
         for e in range(E)], axis=1)
    w1sum_ref[...] = jnp.sum(w1f_ref[...], axis=0, keepdims=True)


def _moe_body(x0, x1, x2, x3, w1f_ref, w2f_ref, gx_ref, w1s_ref, o_ref,
              xbf_ref, *, L, BB):
    xrefs = (x0, x1, x2, x3)
    Q = BB // 4
    RC = 16
    N = o_ref.shape[2]
    for i in range(BB):
        xr, iq = xrefs[i // Q], i % Q

        def step(t, carry, xr=xr, iq=iq):
            a1, a2 = carry
            row = xr[iq, pl.ds(t * RC, RC), :]
            xbf_ref[pl.ds(t * RC, RC), :] = row.astype(jnp.bfloat16)
            return a1 + row, a2 + row * row

        z = jnp.zeros((RC, N), dtype=jnp.float32)
        a1, a2 = jax.lax.fori_loop(0, L // RC, step, (z, z))
        s1 = jnp.sum(a1, axis=0, keepdims=True)
        s2 = jnp.sum(a2, axis=0, keepdims=True)
        mean = s1 * (1.0 / L)
        var = (s2 - mean * s1) * (1.0 / (L - 1))
        std = jnp.sqrt(var) + 1e-6
        rstd = 1.0 / std
        g = jax.lax.dot_general(
            xbf_ref[...], w1f_ref[...],
            (((0,), (0,)), ((), ())), preferred_element_type=jnp.float32,
        )
        mean_c = jnp.transpose(mean)
        rstd_c = jnp.transpose(rstd)
        hg = (g - mean_c * w1s_ref[...]) * (rstd_c * gx_ref[...])
        out_t = jax.lax.dot_general(
            w2f_ref[...], hg.astype(jnp.bfloat16),
            (((0,), (1,)), ((), ())), preferred_element_type=jnp.float32,
        )
        o_ref[i] = out_t * std + mean


def kernel(x, chan_emb, Wr1, br1, Wr2, br2, W1, W2):
    B, L, N = x.shape
    E, _, R = W1.shape
    O = W2.shape[2]
    ER = E * R

    w1f = jnp.transpose(W1, (1, 0, 2)).reshape(L, ER)
    w2f = W2.reshape(ER, O)

    gates_ex, w1sum = pl.pallas_call(
        functools.partial(_router_body, R=R, E=E),
        out_shape=(
            jax.ShapeDtypeStruct((N, ER), jnp.float32),
            jax.ShapeDtypeStruct((1, ER), jnp.float32),
        ),
    )(chan_emb, Wr1, br1.reshape(1, -1), Wr2, br2.reshape(1, -1), w1f)

    BB = 8
    Q = BB // 4
    out = pl.pallas_call(
        functools.partial(_moe_body, L=L, BB=BB),
        grid=(B // BB,),
        in_specs=[
            pl.BlockSpec((Q, L, N), lambda b, j=j: (4 * b + j, 0, 0))
            for j in range(4)
        ] + [
            pl.BlockSpec((L, ER), lambda b: (0, 0)),
            pl.BlockSpec((ER, O), lambda b: (0, 0)),
            pl.BlockSpec((N, ER), lambda b: (0, 0)),
            pl.BlockSpec((1, ER), lambda b: (0, 0)),
        ],
        out_specs=pl.BlockSpec((BB, O, N), lambda b: (b, 0, 0)),
        out_shape=jax.ShapeDtypeStruct((B, O, N), jnp.float32),
        scratch_shapes=[
            pltpu.VMEM((L, N), jnp.bfloat16),
        ],
    )(x, x, x, x, w1f.astype(jnp.bfloat16), w2f.astype(jnp.bfloat16),
      gates_ex, w1sum)
    return out


# speedup vs baseline: 1.5078x; 1.5078x over previous
import functools

import jax
import jax.numpy as jnp
from jax.experimental import pallas as pl


def _router_body(chan_emb_ref, wr1_ref, br1_ref, wr2_ref, br2_ref, w1f_ref,
                 w2f_ref, gates_ref, pm_ref, *, R, E):
    hidden = jnp.maximum(
        jax.lax.dot_general(
            chan_emb_ref[...], wr1_ref[...],
            (((1,), (0,)), ((), ())), preferred_element_type=jnp.float32,
        ) + br1_ref[...],
        0.0,
    )
    logits = jax.lax.dot_general(
        hidden, wr2_ref[...],
        (((1,), (0,)), ((), ())), preferred_element_type=jnp.float32,
    ) + br2_ref[...]
    m = jnp.max(logits, axis=-1, keepdims=True)
    ex = jnp.exp(logits - m)
    gates = ex / jnp.sum(ex, axis=-1, keepdims=True)
    gx = jnp.concatenate(
        [jnp.broadcast_to(gates[:, e:e + 1], gates.shape[:1] + (R,))
         for e in range(E)], axis=1)
    gates_ref[...] = gx
    w1s = jnp.sum(w1f_ref[...], axis=0, keepdims=True)
    p = jax.lax.dot_general(
        w2f_ref[...], gx * w1s,
        (((0,), (1,)), ((), ())), preferred_element_type=jnp.float32,
    )
    pm_ref[...] = 1.0 - p


def _moe_body(x0, x1, x2, x3, w1fe_ref, w2f_ref, gx_ref, pm_ref, o_ref,
              *, L, BB, ER):
    xrefs = (x0, x1, x2, x3)
    Q = BB // 4
    for i in range(BB):
        xb = xrefs[i // Q][i % Q]
        g_ext = jax.lax.dot_general(
            xb.astype(jnp.bfloat16), w1fe_ref[...],
            (((0,), (0,)), ((), ())), preferred_element_type=jnp.float32,
        )
        hg = g_ext[:, :ER] * gx_ref[...]
        mean_row = jnp.transpose(
            g_ext[:, ER:ER + 1]) * (1.0 / L)
        d = jax.lax.dot_general(
            w2f_ref[...], hg.astype(jnp.bfloat16),
            (((0,), (1,)), ((), ())), preferred_element_type=jnp.float32,
        )
        o_ref[i] = d + pm_ref[...] * mean_row


def kernel(x, chan_emb, Wr1, br1, Wr2, br2, W1, W2):
    B, L, N = x.shape
    E, _, R = W1.shape
    O = W2.shape[2]
    ER = E * R

    w1f = jnp.transpose(W1, (1, 0, 2)).reshape(L, ER)
    w2f = W2.reshape(ER, O)

    gates_ex, pm = pl.pallas_call(
        functools.partial(_router_body, R=R, E=E),
        out_shape=(
            jax.ShapeDtypeStruct((N, ER), jnp.float32),
            jax.ShapeDtypeStruct((O, N), jnp.float32),
        ),
    )(chan_emb, Wr1, br1.reshape(1, -1), Wr2, br2.reshape(1, -1), w1f, w2f)

    w1fe = jnp.concatenate(
        [w1f, jnp.ones((L, 1), jnp.float32),
         jnp.zeros((L, 127), jnp.float32)], axis=1).astype(jnp.bfloat16)

    BB = 8
    Q = BB // 4
    out = pl.pallas_call(
        functools.partial(_moe_body, L=L, BB=BB, ER=ER),
        grid=(B // BB,),
        in_specs=[
            pl.BlockSpec((Q, L, N), lambda b, j=j: (4 * b + j, 0, 0))
            for j in range(4)
        ] + [
            pl.BlockSpec((L, ER + 128), lambda b: (0, 0)),
            pl.BlockSpec((ER, O), lambda b: (0, 0)),
            pl.BlockSpec((N, ER), lambda b: (0, 0)),
            pl.BlockSpec((O, N), lambda b: (0, 0)),
        ],
        out_specs=pl.BlockSpec((BB, O, N), lambda b: (b, 0, 0)),
        out_shape=jax.ShapeDtypeStruct((B, O, N), jnp.float32),
    )(x, x, x, x, w1fe, w2f.astype(jnp.bfloat16), gates_ex, pm)
    return out


# speedup vs baseline: 1.5086x; 1.0006x over previous
import functools

import jax
import jax.numpy as jnp
from jax.experimental import pallas as pl
from jax.experimental.pallas import tpu as pltpu


def _router_body(chan_emb_ref, wr1_ref, br1_ref, wr2_ref, br2_ref, w1f_ref,
                 w2f_ref, gates_ref, pm_ref, *, R, E):
    hidden = jnp.maximum(
        jax.lax.dot_general(
            chan_emb_ref[...], wr1_ref[...],
            (((1,), (0,)), ((), ())), preferred_element_type=jnp.float32,
        ) + br1_ref[...],
        0.0,
    )
    logits = jax.lax.dot_general(
        hidden, wr2_ref[...],
        (((1,), (0,)), ((), ())), preferred_element_type=jnp.float32,
    ) + br2_ref[...]
    m = jnp.max(logits, axis=-1, keepdims=True)
    ex = jnp.exp(logits - m)
    gates = ex / jnp.sum(ex, axis=-1, keepdims=True)
    gx = jnp.concatenate(
        [jnp.broadcast_to(gates[:, e:e + 1], gates.shape[:1] + (R,))
         for e in range(E)], axis=1)
    gates_ref[...] = gx
    w1s = jnp.sum(w1f_ref[...], axis=0, keepdims=True)
    p = jax.lax.dot_general(
        w2f_ref[...], gx * w1s,
        (((0,), (1,)), ((), ())), preferred_element_type=jnp.float32,
    )
    pm_ref[...] = 1.0 - p


def _moe_body(x0, x1, x2, x3, w1fe_ref, w2f_ref, gx_ref, pm_ref, o_ref,
              *, L, BB, ER):
    xrefs = (x0, x1, x2, x3)
    Q = BB // 4
    for i in range(BB):
        xb = xrefs[i // Q][i % Q]
        g_ext = jax.lax.dot_general(
            xb.astype(jnp.bfloat16), w1fe_ref[...],
            (((0,), (0,)), ((), ())), preferred_element_type=jnp.float32,
        )
        hg = g_ext[:, :ER] * gx_ref[...]
        mean_row = jnp.transpose(
            g_ext[:, ER:ER + 1]) * (1.0 / L)
        d = jax.lax.dot_general(
            w2f_ref[...], hg.astype(jnp.bfloat16),
            (((0,), (1,)), ((), ())), preferred_element_type=jnp.float32,
        )
        o_ref[i] = d + pm_ref[...] * mean_row


def kernel(x, chan_emb, Wr1, br1, Wr2, br2, W1, W2):
    B, L, N = x.shape
    E, _, R = W1.shape
    O = W2.shape[2]
    ER = E * R

    w1f = jnp.transpose(W1, (1, 0, 2)).reshape(L, ER)
    w2f = W2.reshape(ER, O)

    gates_ex, pm = pl.pallas_call(
        functools.partial(_router_body, R=R, E=E),
        out_shape=(
            jax.ShapeDtypeStruct((N, ER), jnp.float32),
            jax.ShapeDtypeStruct((O, N), jnp.float32),
        ),
    )(chan_emb, Wr1, br1.reshape(1, -1), Wr2, br2.reshape(1, -1), w1f, w2f)

    w1fe = jnp.concatenate(
        [w1f, jnp.ones((L, 1), jnp.float32),
         jnp.zeros((L, 127), jnp.float32)], axis=1).astype(jnp.bfloat16)

    BB = 8
    Q = BB // 4
    out = pl.pallas_call(
        functools.partial(_moe_body, L=L, BB=BB, ER=ER),
        grid=(B // BB,),
        in_specs=[
            pl.BlockSpec((Q, L, N), lambda b, j=j: (4 * b + j, 0, 0))
            for j in range(4)
        ] + [
            pl.BlockSpec((L, ER + 128), lambda b: (0, 0)),
            pl.BlockSpec((ER, O), lambda b: (0, 0)),
            pl.BlockSpec((N, ER), lambda b: (0, 0)),
            pl.BlockSpec((O, N), lambda b: (0, 0)),
        ],
        out_specs=pl.BlockSpec((BB, O, N), lambda b: (b, 0, 0)),
        out_shape=jax.ShapeDtypeStruct((B, O, N), jnp.float32),
        compiler_params=pltpu.CompilerParams(
            dimension_semantics=("parallel",)),
    )(x, x, x, x, w1fe, w2f.astype(jnp.bfloat16), gates_ex, pm)
    return out


# speedup vs baseline: 1.5288x; 1.0134x over previous
import functools

import jax
import jax.numpy as jnp
from jax.experimental import pallas as pl


def _router_body(chan_emb_ref, wr1_ref, br1_ref, wr2_ref, br2_ref, w1f_ref,
                 gates_ref, w1sum_ref, *, R, E):
    hidden = jnp.maximum(
        jax.lax.dot_general(
            chan_emb_ref[...], wr1_ref[...],
            (((1,), (0,)), ((), ())), preferred_element_type=jnp.float32,
        ) + br1_ref[...],
        0.0,
    )
    logits = jax.lax.dot_general(
        hidden, wr2_ref[...],
        (((1,), (0,)), ((), ())), preferred_element_type=jnp.float32,
    ) + br2_ref[...]
    m = jnp.max(logits, axis=-1, keepdims=True)
    ex = jnp.exp(logits - m)
    gates = ex / jnp.sum(ex, axis=-1, keepdims=True)
    gates_ref[...] = jnp.concatenate(
        [jnp.broadcast_to(gates[:, e:e + 1], gates.shape[:1] + (R,))
         for e in range(E)], axis=1)
    w1sum_ref[...] = jnp.sum(w1f_ref[...], axis=0, keepdims=True)


def _moe_body(x_ref, w1f_ref, w2f_ref, gx_ref, w1s_ref, o_ref, *, L, BB):
    for i in range(BB):
        xb = x_ref[i]
        s1 = jnp.sum(xb, axis=0, keepdims=True)
        s2 = jnp.sum(xb * xb, axis=0, keepdims=True)
        mean = s1 / L
        var = (s2 - mean * s1) / (L - 1)
        std = jnp.sqrt(var) + 1e-6
        rstd = 1.0 / std
        g = jax.lax.dot_general(
            xb, w1f_ref[...],
            (((0,), (0,)), ((), ())), preferred_element_type=jnp.float32,
        )
        mean_c = jnp.transpose(mean)
        rstd_c = jnp.transpose(rstd)
        hg = (g - mean_c * w1s_ref[...]) * (rstd_c * gx_ref[...])
        out_t = jax.lax.dot_general(
            w2f_ref[...], hg,
            (((0,), (1,)), ((), ())), preferred_element_type=jnp.float32,
        )
        o_ref[i] = out_t * std + mean


def kernel(x, chan_emb, Wr1, br1, Wr2, br2, W1, W2):
    B, L, N = x.shape
    E, _, R = W1.shape
    O = W2.shape[2]
    ER = E * R

    w1f = jnp.transpose(W1, (1, 0, 2)).reshape(L, ER)
    w2f = W2.reshape(ER, O)

    gates_ex, w1sum = pl.pallas_call(
        functools.partial(_router_body, R=R, E=E),
        out_shape=(
            jax.ShapeDtypeStruct((N, ER), jnp.float32),
            jax.ShapeDtypeStruct((1, ER), jnp.float32),
        ),
    )(chan_emb, Wr1, br1.reshape(1, -1), Wr2, br2.reshape(1, -1), w1f)

    BB = 8
    out = pl.pallas_call(
        functools.partial(_moe_body, L=L, BB=BB),
        grid=(B // BB,),
        in_specs=[
            pl.BlockSpec((BB, L, N), lambda b: (b, 0, 0)),
            pl.BlockSpec((L, ER), lambda b: (0, 0)),
            pl.BlockSpec((ER, O), lambda b: (0, 0)),
            pl.BlockSpec((N, ER), lambda b: (0, 0)),
            pl.BlockSpec((1, ER), lambda b: (0, 0)),
        ],
        out_specs=pl.BlockSpec((BB, O, N), lambda b: (b, 0, 0)),
        out_shape=jax.ShapeDtypeStruct((B, O, N), jnp.float32),
    )(x, w1f, w2f, gates_ex, w1sum)
    return out
